# Initial kernel scaffold; baseline (speedup 1.0000x reference)
import functools

import jax
import jax.numpy as jnp
from jax import lax
from jax.experimental import pallas as pl
from jax.experimental.pallas import tpu as pltpu
from jax.experimental.pallas import tpu_sc as plsc

N_U = 50000
N_I = 50000
N = N_U + N_I
D = 32
E = 1600000
LAYERS = 3

NC = 2
NS = 16
HALF = N // NC
ACC_ROWS = 51200
TRASH0 = HALF
TRASH_MASK = 1023

CHUNK = 128
SEG = 2048
SEG_ROWS = SEG // CHUNK
SEGS = -(-E // (NS * SEG))
E_PAD = SEGS * NS * SEG
TILE_EROWS = SEGS * SEG_ROWS

R2 = HALF // NS
C2 = 125
NCH2 = R2 // C2
G2 = C2 * D // 16

_MESH = plsc.VectorSubcoreMesh(
    core_axis_name="c", subcore_axis_name="s", num_cores=NC, num_subcores=NS)

_F32 = jnp.float32
_I32 = jnp.int32


def _fill_const(ref, val, ngroups):
    vec = jnp.full((16,), val, _F32)

    def body(v, carry):
        ref[v // 2, pl.ds((v % 2) * 16, 16)] = vec
        return carry

    lax.fori_loop(0, ngroups, body, 0)


def _zero_acc(acc, zrow, s):
    _fill_const(zrow, 0.0, 256)
    rows_per_tile = ACC_ROWS // NS

    def body(m, carry):
        pltpu.sync_copy(zrow, acc.at[pl.ds(s * rows_per_tile + m * CHUNK, CHUNK)])
        return carry

    lax.fori_loop(0, rows_per_tile // CHUNK, body, 0)


def _local_dst(dstage, lidx, base, g, iota):
    for k in range(SEG // 16):
        r = k // 8
        col = (k % 8) * 16
        dv = dstage[r, pl.ds(col, 16)]
        dloc = dv - base
        ok = (dloc >= 0) & (dloc < HALF)
        tv = TRASH0 + ((g * SEG + k * 16 + iota) & TRASH_MASK)
        lidx[r, pl.ds(col, 16)] = jnp.where(ok, dloc, tv)


def _deg_body(dst_hbm, deg_hbm, dstage, lidx, ones, zrow, acc, sem):
    c = lax.axis_index("c")
    s = lax.axis_index("s")
    base = c * HALF
    iota = lax.iota(_I32, 16)

    _zero_acc(acc, zrow, s)
    _fill_const(ones, 1.0, 256)
    plsc.subcore_barrier()

    trow = s * TILE_EROWS

    def seg_body(g, carry):
        pltpu.sync_copy(dst_hbm.at[pl.ds(trow + g * SEG_ROWS, SEG_ROWS)], dstage)
        _local_dst(dstage, lidx, base, g, iota)
        for j in range(SEG_ROWS):
            pltpu.sync_copy(ones, acc.at[lidx.at[j]], add=True)
        return carry

    lax.fori_loop(0, SEGS, seg_body, 0)
    plsc.subcore_barrier()

    def out_body(i, carry):
        lr = s * R2 + i * C2
        pltpu.sync_copy(acc.at[pl.ds(lr, C2)], deg_hbm.at[pl.ds(base + lr, C2)])
        return carry

    lax.fori_loop(0, NCH2, out_body, 0)


_deg_kernel = pl.kernel(
    _deg_body,
    out_type=jax.ShapeDtypeStruct((N, D), _F32),
    mesh=_MESH,
    scratch_types=[
        pltpu.VMEM((SEG_ROWS, CHUNK), _I32),
        pltpu.VMEM((SEG_ROWS, CHUNK), _I32),
        pltpu.VMEM((CHUNK, D), _F32),
        pltpu.VMEM((CHUNK, D), _F32),
        pltpu.VMEM_SHARED((ACC_ROWS, D), _F32),
        pltpu.SemaphoreType.DMA,
    ],
)


def _make_layer(final):
    def body(src_hbm, dst_hbm, z_hbm, dis_hbm, dis2_hbm, s_hbm, x0_hbm,
             *rest):
        if final:
            (out_hbm, sstage, dstage, lidx, rows, zrow,
             abuf, db, d2b, sb, x0b, zb, sob, acc, sem) = rest
        else:
            (zout_hbm, sout_hbm, sstage, dstage, lidx, rows, zrow,
             abuf, db, d2b, sb, x0b, zb, sob, acc, sem) = rest

        c = lax.axis_index("c")
        s = lax.axis_index("s")
        base = c * HALF
        iota = lax.iota(_I32, 16)

        _zero_acc(acc, zrow, s)
        plsc.subcore_barrier()

        trow = s * TILE_EROWS

        def seg_body(g, carry):
            r0 = trow + g * SEG_ROWS
            pltpu.sync_copy(src_hbm.at[pl.ds(r0, SEG_ROWS)], sstage)
            pltpu.sync_copy(dst_hbm.at[pl.ds(r0, SEG_ROWS)], dstage)
            _local_dst(dstage, lidx, base, g, iota)
            for j in range(SEG_ROWS):
                pltpu.async_copy(z_hbm.at[sstage.at[j]], rows, sem).wait()
                pltpu.sync_copy(rows, acc.at[lidx.at[j]], add=True)
            return carry

        lax.fori_loop(0, SEGS, seg_body, 0)
        plsc.subcore_barrier()

        def p2(i, carry):
            lr = s * R2 + i * C2
            gr = base + lr
            pltpu.sync_copy(acc.at[pl.ds(lr, C2)], abuf)
            pltpu.sync_copy(dis_hbm.at[pl.ds(gr, C2)], db)
            pltpu.sync_copy(s_hbm.at[pl.ds(gr, C2)], sb)
            if final:
                pltpu.sync_copy(x0_hbm.at[pl.ds(gr, C2)], x0b)
            else:
                pltpu.sync_copy(dis2_hbm.at[pl.ds(gr, C2)], d2b)

            def cg(v, carry2):
                r = v // 2
                cs = (v % 2) * 16
                a = abuf[r, pl.ds(cs, 16)]
                x = a * db[r, pl.ds(cs, 16)]
                if final:
                    zb[r, pl.ds(cs, 16)] = (
                        x0b[r, pl.ds(cs, 16)] + sb[r, pl.ds(cs, 16)] + x) * 0.25
                else:
                    zb[r, pl.ds(cs, 16)] = a * d2b[r, pl.ds(cs, 16)]
                    sob[r, pl.ds(cs, 16)] = sb[r, pl.ds(cs, 16)] + x
                return carry2

            lax.fori_loop(0, G2, cg, 0)
            if final:
                pltpu.sync_copy(zb, out_hbm.at[pl.ds(gr, C2)])
            else:
                pltpu.sync_copy(zb, zout_hbm.at[pl.ds(gr, C2)])
                pltpu.sync_copy(sob, sout_hbm.at[pl.ds(gr, C2)])
            return carry

        lax.fori_loop(0, NCH2, p2, 0)

    n_out = 1 if final else 2
    return pl.kernel(
        body,
        out_type=tuple(jax.ShapeDtypeStruct((N, D), _F32) for _ in range(n_out)),
        mesh=_MESH,
        scratch_types=[
            pltpu.VMEM((SEG_ROWS, CHUNK), _I32),
            pltpu.VMEM((SEG_ROWS, CHUNK), _I32),
            pltpu.VMEM((SEG_ROWS, CHUNK), _I32),
            pltpu.VMEM((CHUNK, D), _F32),
            pltpu.VMEM((CHUNK, D), _F32),
            pltpu.VMEM((C2, D), _F32),
            pltpu.VMEM((C2, D), _F32),
            pltpu.VMEM((C2, D), _F32),
            pltpu.VMEM((C2, D), _F32),
            pltpu.VMEM((C2, D), _F32),
            pltpu.VMEM((C2, D), _F32),
            pltpu.VMEM((C2, D), _F32),
            pltpu.VMEM_SHARED((ACC_ROWS, D), _F32),
            pltpu.SemaphoreType.DMA,
        ],
    )


_layer_mid = _make_layer(final=False)
_layer_fin = _make_layer(final=True)


def _prep_body(deg_ref, x_ref, dis_ref, dis2_ref, z_ref):
    deg = deg_ref[...]
    x = x_ref[...]
    pos = deg > 0.5
    r = lax.rsqrt(jnp.maximum(deg, 1.0))
    dis = jnp.where(pos, r, 0.0)
    dis_ref[...] = dis
    dis2_ref[...] = jnp.where(pos, r * r, 0.0)
    z_ref[...] = x * dis


_PREP_BLOCK = 1000
_prep = pl.pallas_call(
    _prep_body,
    grid=(N // _PREP_BLOCK,),
    in_specs=[pl.BlockSpec((_PREP_BLOCK, D), lambda i: (i, 0))] * 2,
    out_specs=[pl.BlockSpec((_PREP_BLOCK, D), lambda i: (i, 0))] * 3,
    out_shape=[jax.ShapeDtypeStruct((N, D), _F32)] * 3,
)


def kernel(edge_index, user_weight, item_weight):
    src = edge_index[0]
    dst = edge_index[1]
    pad = E_PAD - E
    src2d = jnp.concatenate(
        [src, jnp.zeros((pad,), _I32)]).reshape(E_PAD // CHUNK, CHUNK)
    dst2d = jnp.concatenate(
        [dst, jnp.full((pad,), -1, _I32)]).reshape(E_PAD // CHUNK, CHUNK)
    x0 = jnp.concatenate([user_weight, item_weight], axis=0)

    deg32 = _deg_kernel(dst2d)
    dis32, dis2_32, z = _prep(deg32, x0)
    s = jnp.zeros_like(x0)
    for _ in range(LAYERS - 1):
        z, s = _layer_mid(src2d, dst2d, z, dis32, dis2_32, s, x0)
    out = _layer_fin(src2d, dst2d, z, dis32, dis2_32, s, x0)
    return out[:N_U], out[N_U:]


# speedup vs baseline: 12.4838x; 12.4838x over previous
import jax
import jax.numpy as jnp
from jax import lax
from jax.experimental import pallas as pl
from jax.experimental.pallas import tpu as pltpu
from jax.experimental.pallas import tpu_sc as plsc

N_U = 50000
N_I = 50000
N = N_U + N_I
D = 32
E = 1600000
LAYERS = 3

NC = 2
NS = 16
HALF = N // NC
ACC_ROWS = 51200
TRASH0 = HALF
TRASH_MASK = 1023

CHUNK = 128
SEG = 1024
SEG_ROWS = SEG // CHUNK
SEGS = -(-E // (NS * SEG))
E_PAD = SEGS * NS * SEG
TILE_EROWS = SEGS * SEG_ROWS

R2 = HALF // NS
C2 = 125
NCH2 = R2 // C2
G2 = C2 * D // 16

_MESH = plsc.VectorSubcoreMesh(
    core_axis_name="c", subcore_axis_name="s", num_cores=NC, num_subcores=NS)

_F32 = jnp.float32
_I32 = jnp.int32


def _fill_const(ref, val, ngroups):
    vec = jnp.full((16,), val, _F32)

    def body(v, carry):
        ref[v // 2, pl.ds((v % 2) * 16, 16)] = vec
        return carry

    lax.fori_loop(0, ngroups, body, 0)


def _zero_acc(acc, zrow, s):
    rows_per_tile = ACC_ROWS // NS

    def body(m, carry):
        pltpu.sync_copy(zrow, acc.at[pl.ds(s * rows_per_tile + m * CHUNK, CHUNK)])
        return carry

    lax.fori_loop(0, rows_per_tile // CHUNK, body, 0)


def _local_dst(dstage, lidx, base, g, iota):
    for k in range(SEG // 16):
        r = k // 8
        col = (k % 8) * 16
        dv = dstage[r, pl.ds(col, 16)]
        dloc = dv - base
        ok = (dloc >= 0) & (dloc < HALF)
        tv = TRASH0 + ((g * SEG + k * 16 + iota) & TRASH_MASK)
        lidx[r, pl.ds(col, 16)] = jnp.where(ok, dloc, tv)


def _deg_body(dst_hbm, deg_hbm, dstage, lidx, ones, acc, sem):
    c = lax.axis_index("c")
    s = lax.axis_index("s")
    base = c * HALF
    iota = lax.iota(_I32, 16)

    _fill_const(ones, 0.0, CHUNK * D // 16)
    _zero_acc(acc, ones, s)
    _fill_const(ones, 1.0, CHUNK * D // 16)
    plsc.subcore_barrier()

    trow = s * TILE_EROWS

    def seg_body(g, carry):
        pltpu.sync_copy(dst_hbm.at[pl.ds(trow + g * SEG_ROWS, SEG_ROWS)], dstage)
        _local_dst(dstage, lidx, base, g, iota)
        for j in range(SEG_ROWS):
            pltpu.sync_copy(ones, acc.at[lidx.at[j]], add=True)
        return carry

    lax.fori_loop(0, SEGS, seg_body, 0)
    plsc.subcore_barrier()

    def out_body(i, carry):
        lr = s * R2 + i * C2
        pltpu.sync_copy(acc.at[pl.ds(lr, C2)], deg_hbm.at[pl.ds(base + lr, C2)])
        return carry

    lax.fori_loop(0, NCH2, out_body, 0)


_deg_kernel = pl.kernel(
    _deg_body,
    out_type=jax.ShapeDtypeStruct((N, D), _F32),
    mesh=_MESH,
    compiler_params=pltpu.CompilerParams(use_tc_tiling_on_sc=False),
    scratch_types=[
        pltpu.VMEM((SEG_ROWS, CHUNK), _I32),
        pltpu.VMEM((SEG_ROWS, CHUNK), _I32),
        pltpu.VMEM((CHUNK, D), _F32),
        pltpu.VMEM_SHARED((ACC_ROWS, D), _F32),
        pltpu.SemaphoreType.DMA,
    ],
)


def _make_layer(final):
    def body(src_hbm, dst_hbm, z_hbm, dis_hbm, dis2_hbm, s_hbm, *rest):
        if final:
            (out_hbm, sstage, dstage, lidx, rows,
             abuf, db, d2b, sb, acc, sem) = rest
        else:
            (zout_hbm, sout_hbm, sstage, dstage, lidx, rows,
             abuf, db, d2b, sb, acc, sem) = rest

        c = lax.axis_index("c")
        s = lax.axis_index("s")
        base = c * HALF
        iota = lax.iota(_I32, 16)

        _fill_const(rows, 0.0, CHUNK * D // 16)
        _zero_acc(acc, rows, s)
        plsc.subcore_barrier()

        trow = s * TILE_EROWS

        def seg_body(g, carry):
            r0 = trow + g * SEG_ROWS
            pltpu.sync_copy(src_hbm.at[pl.ds(r0, SEG_ROWS)], sstage)
            pltpu.sync_copy(dst_hbm.at[pl.ds(r0, SEG_ROWS)], dstage)
            _local_dst(dstage, lidx, base, g, iota)
            for j in range(SEG_ROWS):
                pltpu.async_copy(z_hbm.at[sstage.at[j]], rows, sem).wait()
                pltpu.sync_copy(rows, acc.at[lidx.at[j]], add=True)
            return carry

        lax.fori_loop(0, SEGS, seg_body, 0)
        plsc.subcore_barrier()

        def p2(i, carry):
            lr = s * R2 + i * C2
            gr = base + lr
            pltpu.sync_copy(acc.at[pl.ds(lr, C2)], abuf)
            pltpu.sync_copy(dis_hbm.at[pl.ds(gr, C2)], db)
            pltpu.sync_copy(s_hbm.at[pl.ds(gr, C2)], sb)
            if not final:
                pltpu.sync_copy(dis2_hbm.at[pl.ds(gr, C2)], d2b)

            def cg(v, carry2):
                r = v // 2
                cs = (v % 2) * 16
                a = abuf[r, pl.ds(cs, 16)]
                x = a * db[r, pl.ds(cs, 16)]
                if final:
                    abuf[r, pl.ds(cs, 16)] = (sb[r, pl.ds(cs, 16)] + x) * 0.25
                else:
                    abuf[r, pl.ds(cs, 16)] = a * d2b[r, pl.ds(cs, 16)]
                    sb[r, pl.ds(cs, 16)] = sb[r, pl.ds(cs, 16)] + x
                return carry2

            lax.fori_loop(0, G2, cg, 0)
            if final:
                pltpu.sync_copy(abuf, out_hbm.at[pl.ds(gr, C2)])
            else:
                pltpu.sync_copy(abuf, zout_hbm.at[pl.ds(gr, C2)])
                pltpu.sync_copy(sb, sout_hbm.at[pl.ds(gr, C2)])
            return carry

        lax.fori_loop(0, NCH2, p2, 0)

    n_out = 1 if final else 2
    return pl.kernel(
        body,
        out_type=tuple(jax.ShapeDtypeStruct((N, D), _F32) for _ in range(n_out)),
        mesh=_MESH,
        compiler_params=pltpu.CompilerParams(use_tc_tiling_on_sc=False),
        scratch_types=[
            pltpu.VMEM((SEG_ROWS, CHUNK), _I32),
            pltpu.VMEM((SEG_ROWS, CHUNK), _I32),
            pltpu.VMEM((SEG_ROWS, CHUNK), _I32),
            pltpu.VMEM((CHUNK, D), _F32),
            pltpu.VMEM((C2, D), _F32),
            pltpu.VMEM((C2, D), _F32),
            pltpu.VMEM((C2, D), _F32),
            pltpu.VMEM((C2, D), _F32),
            pltpu.VMEM_SHARED((ACC_ROWS, D), _F32),
            pltpu.SemaphoreType.DMA,
        ],
    )


_layer_mid = _make_layer(final=False)
_layer_fin = _make_layer(final=True)


def _prep_body(deg_ref, x_ref, dis_ref, dis2_ref, z_ref):
    deg = deg_ref[...]
    x = x_ref[...]
    pos = deg > 0.5
    r = lax.rsqrt(jnp.maximum(deg, 1.0))
    dis = jnp.where(pos, r, 0.0)
    dis_ref[...] = dis
    dis2_ref[...] = jnp.where(pos, r * r, 0.0)
    z_ref[...] = x * dis


_PREP_BLOCK = 1000
_prep = pl.pallas_call(
    _prep_body,
    grid=(N // _PREP_BLOCK,),
    in_specs=[pl.BlockSpec((_PREP_BLOCK, D), lambda i: (i, 0))] * 2,
    out_specs=[pl.BlockSpec((_PREP_BLOCK, D), lambda i: (i, 0))] * 3,
    out_shape=[jax.ShapeDtypeStruct((N, D), _F32)] * 3,
)


def kernel(edge_index, user_weight, item_weight):
    src = edge_index[0]
    dst = edge_index[1]
    pad = E_PAD - E
    src2d = jnp.concatenate(
        [src, jnp.zeros((pad,), _I32)]).reshape(E_PAD // CHUNK, CHUNK)
    dst2d = jnp.concatenate(
        [dst, jnp.full((pad,), -1, _I32)]).reshape(E_PAD // CHUNK, CHUNK)
    x0 = jnp.concatenate([user_weight, item_weight], axis=0)

    deg32 = _deg_kernel(dst2d)
    dis32, dis2_32, z = _prep(deg32, x0)
    s = x0
    for _ in range(LAYERS - 1):
        z, s = _layer_mid(src2d, dst2d, z, dis32, dis2_32, s)
    (out,) = _layer_fin(src2d, dst2d, z, dis32, dis2_32, s)
    return out[:N_U], out[N_U:]


# speedup vs baseline: 18.1169x; 1.4512x over previous
import jax
import jax.numpy as jnp
from jax import lax
from jax.experimental import pallas as pl
from jax.experimental.pallas import tpu as pltpu
from jax.experimental.pallas import tpu_sc as plsc

N_U = 50000
N_I = 50000
N = N_U + N_I
D = 32
E = 1600000
LAYERS = 3

NC = 2
NS = 16
HALF = N // NC
ACC_ROWS = 51200
TRASH0 = HALF
TRASH_MASK = 1023

CHUNK = 128
SEG = 1024
SEG_ROWS = SEG // CHUNK
SEGS = -(-E // (NS * SEG))
E_PAD = SEGS * NS * SEG
TILE_EROWS = SEGS * SEG_ROWS

NBUF = 4

R2 = HALF // NS
C2 = 125
NCH2 = R2 // C2

_MESH = plsc.VectorSubcoreMesh(
    core_axis_name="c", subcore_axis_name="s", num_cores=NC, num_subcores=NS)

_F32 = jnp.float32
_I32 = jnp.int32


def _fill_const(ref, val, ngroups):
    vec = jnp.full((16,), val, _F32)

    def body(v, carry):
        ref[v // 2, pl.ds((v % 2) * 16, 16)] = vec
        return carry

    lax.fori_loop(0, ngroups, body, 0)


def _zero_acc(acc, zrow, s, sem):
    rows_per_tile = ACC_ROWS // NS
    descs = [
        pltpu.async_copy(
            zrow, acc.at[pl.ds(s * rows_per_tile + m * CHUNK, CHUNK)], sem)
        for m in range(rows_per_tile // CHUNK)
    ]
    for d in descs:
        d.wait()


def _dump_acc(acc, out_hbm, s, base, sem):
    descs = [
        pltpu.async_copy(
            acc.at[pl.ds(s * R2 + i * C2, C2)],
            out_hbm.at[pl.ds(base + s * R2 + i * C2, C2)], sem)
        for i in range(NCH2)
    ]
    for d in descs:
        d.wait()


def _local_dst(dstage, lidx, base, g, iota):
    for k in range(SEG // 16):
        r = k // 8
        col = (k % 8) * 16
        dv = dstage[r, pl.ds(col, 16)]
        dloc = dv - base
        ok = (dloc >= 0) & (dloc < HALF)
        tv = TRASH0 + ((g * SEG + k * 16 + iota) & TRASH_MASK)
        lidx[r, pl.ds(col, 16)] = jnp.where(ok, dloc, tv)


def _deg_body(dst_hbm, deg_hbm, dstage, lidx, ones, acc, sem):
    c = lax.axis_index("c")
    s = lax.axis_index("s")
    base = c * HALF
    iota = lax.iota(_I32, 16)

    _fill_const(ones, 0.0, CHUNK * D // 16)
    _zero_acc(acc, ones, s, sem)
    _fill_const(ones, 1.0, CHUNK * D // 16)
    plsc.subcore_barrier()

    trow = s * TILE_EROWS

    def seg_body(g, carry):
        pltpu.sync_copy(dst_hbm.at[pl.ds(trow + g * SEG_ROWS, SEG_ROWS)], dstage)
        _local_dst(dstage, lidx, base, g, iota)
        descs = [
            pltpu.async_copy(ones, acc.at[lidx.at[j]], sem, add=True)
            for j in range(SEG_ROWS)
        ]
        for d in descs:
            d.wait()
        return carry

    lax.fori_loop(0, SEGS, seg_body, 0)
    plsc.subcore_barrier()
    _dump_acc(acc, deg_hbm, s, base, sem)


_deg_kernel = pl.kernel(
    _deg_body,
    out_type=jax.ShapeDtypeStruct((N, D), _F32),
    mesh=_MESH,
    compiler_params=pltpu.CompilerParams(use_tc_tiling_on_sc=False),
    scratch_types=[
        pltpu.VMEM((SEG_ROWS, CHUNK), _I32),
        pltpu.VMEM((SEG_ROWS, CHUNK), _I32),
        pltpu.VMEM((CHUNK, D), _F32),
        pltpu.VMEM_SHARED((ACC_ROWS, D), _F32),
        pltpu.SemaphoreType.DMA,
    ],
)


def _scatter_body(src_hbm, dst_hbm, z_hbm, acc_out_hbm,
                  sstage, dstage, lidx, rows, acc, dsem, *sems):
    gsem = sems[:NBUF]
    ssem = sems[NBUF:]

    c = lax.axis_index("c")
    s = lax.axis_index("s")
    base = c * HALF
    iota = lax.iota(_I32, 16)

    _fill_const(rows[0], 0.0, CHUNK * D // 16)
    _zero_acc(acc, rows[0], s, dsem)
    plsc.subcore_barrier()

    trow = s * TILE_EROWS

    def seg_body(g, carry):
        r0 = trow + g * SEG_ROWS
        pltpu.sync_copy(src_hbm.at[pl.ds(r0, SEG_ROWS)], sstage)
        pltpu.sync_copy(dst_hbm.at[pl.ds(r0, SEG_ROWS)], dstage)
        _local_dst(dstage, lidx, base, g, iota)

        gd = {}
        sd = {}
        for j in range(min(NBUF - 1, SEG_ROWS)):
            gd[j] = pltpu.async_copy(
                z_hbm.at[sstage.at[j]], rows[j % NBUF], gsem[j % NBUF])
        for j in range(SEG_ROWS):
            slot = j % NBUF
            gd[j].wait()
            sd[j] = pltpu.async_copy(
                rows[slot], acc.at[lidx.at[j]], ssem[slot], add=True)
            nj = j + NBUF - 1
            if nj < SEG_ROWS:
                nslot = nj % NBUF
                if nj - NBUF >= 0:
                    sd[nj - NBUF].wait()
                gd[nj] = pltpu.async_copy(
                    z_hbm.at[sstage.at[nj]], rows[nslot], gsem[nslot])
        for j in range(max(0, SEG_ROWS - NBUF), SEG_ROWS):
            sd[j].wait()
        return carry

    lax.fori_loop(0, SEGS, seg_body, 0)
    plsc.subcore_barrier()
    _dump_acc(acc, acc_out_hbm, s, base, dsem)


_scatter_kernel = pl.kernel(
    _scatter_body,
    out_type=jax.ShapeDtypeStruct((N, D), _F32),
    mesh=_MESH,
    compiler_params=pltpu.CompilerParams(use_tc_tiling_on_sc=False),
    scratch_types=(
        [
            pltpu.VMEM((SEG_ROWS, CHUNK), _I32),
            pltpu.VMEM((SEG_ROWS, CHUNK), _I32),
            pltpu.VMEM((SEG_ROWS, CHUNK), _I32),
            [pltpu.VMEM((CHUNK, D), _F32)] * NBUF,
            pltpu.VMEM_SHARED((ACC_ROWS, D), _F32),
        ]
        + [pltpu.SemaphoreType.DMA] * (1 + 2 * NBUF)
    ),
)


_PREP_BLOCK = 1000
_EW_GRID = (N // _PREP_BLOCK,)
_EW_SPEC = pl.BlockSpec((_PREP_BLOCK, D), lambda i: (i, 0))


def _prep_body(deg_ref, x_ref, dis_ref, dis2_ref, z_ref):
    deg = deg_ref[...]
    x = x_ref[...]
    pos = deg > 0.5
    r = lax.rsqrt(jnp.maximum(deg, 1.0))
    dis = jnp.where(pos, r, 0.0)
    dis_ref[...] = dis
    dis2_ref[...] = jnp.where(pos, r * r, 0.0)
    z_ref[...] = x * dis


_prep = pl.pallas_call(
    _prep_body,
    grid=_EW_GRID,
    in_specs=[_EW_SPEC] * 2,
    out_specs=[_EW_SPEC] * 3,
    out_shape=[jax.ShapeDtypeStruct((N, D), _F32)] * 3,
)


def _post_body(acc_ref, dis_ref, dis2_ref, s_ref, z_ref, sout_ref):
    a = acc_ref[...]
    z_ref[...] = a * dis2_ref[...]
    sout_ref[...] = s_ref[...] + a * dis_ref[...]


_post = pl.pallas_call(
    _post_body,
    grid=_EW_GRID,
    in_specs=[_EW_SPEC] * 4,
    out_specs=[_EW_SPEC] * 2,
    out_shape=[jax.ShapeDtypeStruct((N, D), _F32)] * 2,
)


def _fin_body(acc_ref, dis_ref, s_ref, out_ref):
    out_ref[...] = (s_ref[...] + acc_ref[...] * dis_ref[...]) * 0.25


_fin = pl.pallas_call(
    _fin_body,
    grid=_EW_GRID,
    in_specs=[_EW_SPEC] * 3,
    out_specs=_EW_SPEC,
    out_shape=jax.ShapeDtypeStruct((N, D), _F32),
)


def kernel(edge_index, user_weight, item_weight):
    src = edge_index[0]
    dst = edge_index[1]
    pad = E_PAD - E
    src2d = jnp.concatenate(
        [src, jnp.zeros((pad,), _I32)]).reshape(E_PAD // CHUNK, CHUNK)
    dst2d = jnp.concatenate(
        [dst, jnp.full((pad,), -1, _I32)]).reshape(E_PAD // CHUNK, CHUNK)
    x0 = jnp.concatenate([user_weight, item_weight], axis=0)

    deg32 = _deg_kernel(dst2d)
    dis32, dis2_32, z = _prep(deg32, x0)
    s = x0
    for _ in range(LAYERS - 1):
        acc = _scatter_kernel(src2d, dst2d, z)
        z, s = _post(acc, dis32, dis2_32, s)
    acc = _scatter_kernel(src2d, dst2d, z)
    out = _fin(acc, dis32, s)
    return out[:N_U], out[N_U:]


# speedup vs baseline: 30.0238x; 1.6572x over previous
import jax
import jax.numpy as jnp
from jax import lax
from jax.experimental import pallas as pl
from jax.experimental.pallas import tpu as pltpu
from jax.experimental.pallas import tpu_sc as plsc

N_U = 50000
N_I = 50000
N = N_U + N_I
D = 32
E = 1600000
LAYERS = 3

NC = 2
NS = 16
HALF = N // NC
ACC_ROWS = 51200
TRASH0 = HALF

CHUNK = 128
SEG = 2048
SEG_ROWS = SEG // CHUNK
SEGS = -(-E // (NS * SEG))
E_PAD = SEGS * NS * SEG
TILE_EROWS = SEGS * SEG_ROWS

CAPR = TILE_EROWS + SEG_ROWS
CBUF = SEG + 2 * CHUNK + 16
DUMP0 = SEG + 2 * CHUNK

NBUF = 6

R2 = HALF // NS
C2 = 125
NCH2 = R2 // C2

_MESH = plsc.VectorSubcoreMesh(
    core_axis_name="c", subcore_axis_name="s", num_cores=NC, num_subcores=NS)

_F32 = jnp.float32
_I32 = jnp.int32
_SC_PARAMS = pltpu.CompilerParams(
    use_tc_tiling_on_sc=False, needs_layout_passes=False)


def _fill_const(ref, val, ngroups):
    vec = jnp.full((16,), val, _F32)

    def body(v, carry):
        ref[v // 2, pl.ds((v % 2) * 16, 16)] = vec
        return carry

    lax.fori_loop(0, ngroups, body, 0)


def _zero_acc(acc, zrow, s, sem):
    rows_per_tile = ACC_ROWS // NS
    descs = [
        pltpu.async_copy(
            zrow, acc.at[pl.ds(s * rows_per_tile + m * CHUNK, CHUNK)], sem)
        for m in range(rows_per_tile // CHUNK)
    ]
    for d in descs:
        d.wait()


def _dump_acc(acc, out_hbm, s, base, sem):
    descs = [
        pltpu.async_copy(
            acc.at[pl.ds(s * R2 + i * C2, C2)],
            out_hbm.at[pl.ds(base + s * R2 + i * C2, C2)], sem)
        for i in range(NCH2)
    ]
    for d in descs:
        d.wait()


def _read_nsegs(pcnt_hbm, lidx, row):
    pltpu.sync_copy(pcnt_hbm.at[row], lidx.at[0])
    return jnp.max(lidx[0, pl.ds(0, 16)])


def _part_body(src_hbm, dst_hbm, psrc_hbm, pldst_hbm, pcnt_hbm,
               sstage, dstage, cbs, cbd, padrow, cntb, fsem, stsem):
    c = lax.axis_index("c")
    s = lax.axis_index("s")
    base = c * HALF
    iota = lax.iota(_I32, 16)
    trow = s * TILE_EROWS
    tb_out = (c * NS + s) * CAPR

    for m in range(8):
        padrow[0, pl.ds(m * 16, 16)] = TRASH0 + m * 16 + iota

    def seg_body(g, carry):
        rem, nrows = carry
        offv = jnp.zeros((16,), _I32) + rem
        d1 = pltpu.async_copy(
            src_hbm.at[pl.ds(trow + g * SEG_ROWS, SEG_ROWS)], sstage, stsem)
        d2 = pltpu.async_copy(
            dst_hbm.at[pl.ds(trow + g * SEG_ROWS, SEG_ROWS)], dstage, stsem)
        d1.wait()
        d2.wait()
        for k in range(SEG // 16):
            r = k // 8
            col = (k % 8) * 16
            sv = sstage[r, pl.ds(col, 16)]
            dv = dstage[r, pl.ds(col, 16)]
            dloc = dv - base
            msk = (dloc >= 0) & (dloc < HALF)
            pos = plsc.cumsum(jnp.where(msk, 1, 0))
            tgt = jnp.where(msk, offv + pos - 1, DUMP0 + iota)
            plsc.store_scatter(cbs, [tgt], sv)
            plsc.store_scatter(cbd, [tgt], dloc)
            offv = offv + plsc.all_reduce_population_count(msk)
        off = jnp.max(offv)
        nfull = off // CHUNK

        def fire(i, carry2):
            pltpu.async_copy(
                cbs.at[pl.ds(i * CHUNK, CHUNK)],
                psrc_hbm.at[tb_out + nrows + i], fsem)
            pltpu.async_copy(
                cbd.at[pl.ds(i * CHUNK, CHUNK)],
                pldst_hbm.at[tb_out + nrows + i], fsem)
            return carry2

        lax.fori_loop(0, nfull, fire, 0)

        def drain(i, carry2):
            pltpu.make_async_copy(
                psrc_hbm.at[tb_out], cbs.at[pl.ds(0, CHUNK)], fsem).wait()
            return carry2

        lax.fori_loop(0, 2 * nfull, drain, 0)

        rem = off - nfull * CHUNK
        for gsh in range(CHUNK // 16):
            src_off = nfull * CHUNK + gsh * 16
            vs = cbs[pl.ds(src_off, 16)]
            vd = cbd[pl.ds(src_off, 16)]
            cbs[pl.ds(gsh * 16, 16)] = vs
            cbd[pl.ds(gsh * 16, 16)] = vd
        return rem, nrows + nfull

    rem, nrows = lax.fori_loop(0, SEGS, seg_body, (0, 0))
    off = rem

    for gsh in range(CHUNK // 16):
        pos = gsh * 16 + iota
        keep = pos < off
        vs = cbs[pl.ds(gsh * 16, 16)]
        vd = cbd[pl.ds(gsh * 16, 16)]
        cbs[pl.ds(gsh * 16, 16)] = jnp.where(keep, vs, 0)
        cbd[pl.ds(gsh * 16, 16)] = jnp.where(keep, vd, TRASH0 + pos - off)

    @pl.when(off > 0)
    def _():
        pltpu.sync_copy(cbs.at[pl.ds(0, CHUNK)], psrc_hbm.at[tb_out + nrows])
        pltpu.sync_copy(cbd.at[pl.ds(0, CHUNK)], pldst_hbm.at[tb_out + nrows])

    nrows = nrows + jnp.where(off > 0, 1, 0)

    npad = (-nrows) % SEG_ROWS
    for m in range(8):
        padrow[0, pl.ds(m * 16, 16)] = TRASH0 + m * 16 + iota

    def padfill(i, carry2):
        pltpu.sync_copy(padrow.at[0], pldst_hbm.at[tb_out + nrows + i])
        pltpu.sync_copy(padrow.at[0], psrc_hbm.at[tb_out + nrows + i])
        return carry2

    lax.fori_loop(0, npad, padfill, 0)
    nsegs = (nrows + npad) // SEG_ROWS
    for m in range(8):
        cntb[pl.ds(m * 16, 16)] = jnp.zeros((16,), _I32) + nsegs
    pltpu.sync_copy(cntb, pcnt_hbm.at[c * NS + s])


_part_kernel = pl.kernel(
    _part_body,
    out_type=(
        jax.ShapeDtypeStruct((NC * NS * CAPR, CHUNK), _I32),
        jax.ShapeDtypeStruct((NC * NS * CAPR, CHUNK), _I32),
        jax.ShapeDtypeStruct((NC * NS, CHUNK), _I32),
    ),
    mesh=_MESH,
    compiler_params=_SC_PARAMS,
    scratch_types=[
        pltpu.VMEM((SEG_ROWS, CHUNK), _I32),
        pltpu.VMEM((SEG_ROWS, CHUNK), _I32),
        pltpu.VMEM((CBUF,), _I32),
        pltpu.VMEM((CBUF,), _I32),
        pltpu.VMEM((1, CHUNK), _I32),
        pltpu.VMEM((CHUNK,), _I32),
        pltpu.SemaphoreType.DMA,
        pltpu.SemaphoreType.DMA,
    ],
)


def _deg_body(pldst_hbm, pcnt_hbm, deg_hbm, lidx, ones, acc, sem):
    c = lax.axis_index("c")
    s = lax.axis_index("s")
    base = c * HALF
    tb_out = (c * NS + s) * CAPR

    _fill_const(ones, 0.0, CHUNK * D // 16)
    _zero_acc(acc, ones, s, sem)
    _fill_const(ones, 1.0, CHUNK * D // 16)
    nsegs = _read_nsegs(pcnt_hbm, lidx, c * NS + s)
    plsc.subcore_barrier()

    def seg_body(g, carry):
        pltpu.sync_copy(pldst_hbm.at[pl.ds(tb_out + g * SEG_ROWS, SEG_ROWS)],
                        lidx)
        descs = [
            pltpu.async_copy(ones, acc.at[lidx.at[j]], sem, add=True)
            for j in range(SEG_ROWS)
        ]
        for d in descs:
            d.wait()
        return carry

    lax.fori_loop(0, nsegs, seg_body, 0)
    plsc.subcore_barrier()
    _dump_acc(acc, deg_hbm, s, base, sem)


_deg_kernel = pl.kernel(
    _deg_body,
    out_type=jax.ShapeDtypeStruct((N, D), _F32),
    mesh=_MESH,
    compiler_params=_SC_PARAMS,
    scratch_types=[
        pltpu.VMEM((SEG_ROWS, CHUNK), _I32),
        pltpu.VMEM((CHUNK, D), _F32),
        pltpu.VMEM_SHARED((ACC_ROWS, D), _F32),
        pltpu.SemaphoreType.DMA,
    ],
)


def _scatter_body(psrc_hbm, pldst_hbm, pcnt_hbm, z_hbm, acc_out_hbm,
                  sstage, lidx, rows, acc, dsem, stsem, *sems):
    gsem = sems[:NBUF]
    ssem = sems[NBUF:]

    c = lax.axis_index("c")
    s = lax.axis_index("s")
    base = c * HALF
    tb_out = (c * NS + s) * CAPR

    _fill_const(rows[0], 0.0, CHUNK * D // 16)
    _zero_acc(acc, rows[0], s, dsem)
    nsegs = _read_nsegs(pcnt_hbm, lidx, c * NS + s)
    plsc.subcore_barrier()

    def seg_body(g, carry):
        r0 = tb_out + g * SEG_ROWS
        d1 = pltpu.async_copy(psrc_hbm.at[pl.ds(r0, SEG_ROWS)], sstage, stsem)
        d2 = pltpu.async_copy(pldst_hbm.at[pl.ds(r0, SEG_ROWS)], lidx, stsem)
        d1.wait()
        d2.wait()

        gd = {}
        sd = {}
        for j in range(min(NBUF - 1, SEG_ROWS)):
            gd[j] = pltpu.async_copy(
                z_hbm.at[sstage.at[j]], rows[j % NBUF], gsem[j % NBUF])
        for j in range(SEG_ROWS):
            slot = j % NBUF
            gd[j].wait()
            sd[j] = pltpu.async_copy(
                rows[slot], acc.at[lidx.at[j]], ssem[slot], add=True)
            nj = j + NBUF - 1
            if nj < SEG_ROWS:
                nslot = nj % NBUF
                if nj - NBUF >= 0:
                    sd[nj - NBUF].wait()
                gd[nj] = pltpu.async_copy(
                    z_hbm.at[sstage.at[nj]], rows[nslot], gsem[nslot])
        for j in range(max(0, SEG_ROWS - NBUF), SEG_ROWS):
            sd[j].wait()
        return carry

    lax.fori_loop(0, nsegs, seg_body, 0)
    plsc.subcore_barrier()
    _dump_acc(acc, acc_out_hbm, s, base, dsem)


_scatter_kernel = pl.kernel(
    _scatter_body,
    out_type=jax.ShapeDtypeStruct((N, D), _F32),
    mesh=_MESH,
    compiler_params=_SC_PARAMS,
    scratch_types=(
        [
            pltpu.VMEM((SEG_ROWS, CHUNK), _I32),
            pltpu.VMEM((SEG_ROWS, CHUNK), _I32),
            [pltpu.VMEM((CHUNK, D), _F32)] * NBUF,
            pltpu.VMEM_SHARED((ACC_ROWS, D), _F32),
        ]
        + [pltpu.SemaphoreType.DMA] * (2 + 2 * NBUF)
    ),
)


_PREP_BLOCK = 1000
_EW_GRID = (N // _PREP_BLOCK,)
_EW_SPEC = pl.BlockSpec((_PREP_BLOCK, D), lambda i: (i, 0))


def _prep_body(deg_ref, x_ref, dis_ref, dis2_ref, z_ref):
    deg = deg_ref[...]
    x = x_ref[...]
    pos = deg > 0.5
    r = lax.rsqrt(jnp.maximum(deg, 1.0))
    dis = jnp.where(pos, r, 0.0)
    dis_ref[...] = dis
    dis2_ref[...] = jnp.where(pos, r * r, 0.0)
    z_ref[...] = x * dis


_prep = pl.pallas_call(
    _prep_body,
    grid=_EW_GRID,
    in_specs=[_EW_SPEC] * 2,
    out_specs=[_EW_SPEC] * 3,
    out_shape=[jax.ShapeDtypeStruct((N, D), _F32)] * 3,
)


def _post_body(acc_ref, dis2_ref, z_ref):
    z_ref[...] = acc_ref[...] * dis2_ref[...]


_post = pl.pallas_call(
    _post_body,
    grid=_EW_GRID,
    in_specs=[_EW_SPEC] * 2,
    out_specs=_EW_SPEC,
    out_shape=jax.ShapeDtypeStruct((N, D), _F32),
)


def _fin_body(x0_ref, dis_ref, a1_ref, a2_ref, a3_ref, out_ref):
    asum = a1_ref[...] + a2_ref[...] + a3_ref[...]
    out_ref[...] = (x0_ref[...] + dis_ref[...] * asum) * 0.25


_fin = pl.pallas_call(
    _fin_body,
    grid=_EW_GRID,
    in_specs=[_EW_SPEC] * 5,
    out_specs=_EW_SPEC,
    out_shape=jax.ShapeDtypeStruct((N, D), _F32),
)


def kernel(edge_index, user_weight, item_weight):
    src = edge_index[0]
    dst = edge_index[1]
    pad = E_PAD - E
    src2d = jnp.concatenate(
        [src, jnp.zeros((pad,), _I32)]).reshape(E_PAD // CHUNK, CHUNK)
    dst2d = jnp.concatenate(
        [dst, jnp.full((pad,), -1, _I32)]).reshape(E_PAD // CHUNK, CHUNK)
    x0 = jnp.concatenate([user_weight, item_weight], axis=0)

    psrc, pldst, pcnt = _part_kernel(src2d, dst2d)
    deg32 = _deg_kernel(pldst, pcnt)
    dis32, dis2_32, z = _prep(deg32, x0)
    accs = []
    for l in range(LAYERS):
        acc = _scatter_kernel(psrc, pldst, pcnt, z)
        accs.append(acc)
        if l < LAYERS - 1:
            z = _post(acc, dis2_32)
    out = _fin(x0, dis32, accs[0], accs[1], accs[2])
    return out[:N_U], out[N_U:]
